# Initial kernel scaffold; baseline (speedup 1.0000x reference)
import jax
import jax.numpy as jnp
from jax.experimental import pallas as pl


def kernel(x, edge_index, edge_weight, W1, b1, W2, b2, W3, b3):
    raise NotImplementedError("write your pallas kernel here")


# speedup vs baseline: 14.1781x; 14.1781x over previous
import functools

import jax
import jax.numpy as jnp
from jax import lax
from jax.experimental import pallas as pl
from jax.experimental.pallas import tpu as pltpu
from jax.experimental.pallas import tpu_sc as plsc

N = 50000
NP = 51200
E = 800000
EP = 819200
STRIPE = NP // 16
BM = 3200
GM = NP // BM
KP = 384

_mesh = plsc.VectorSubcoreMesh(core_axis_name="c", subcore_axis_name="s")


def _deg_body(dst2d, ew2d, dega, degb, dstb, ewb, zbuf, acc):
    ci = lax.axis_index("c")
    si = lax.axis_index("s")

    def zb(i, _):
        zbuf[pl.ds(i * 16, 16)] = jnp.zeros((16,), jnp.float32)
        return 0

    lax.fori_loop(0, STRIPE // 16, zb, 0)
    pltpu.sync_copy(zbuf, acc.at[pl.ds(si * STRIPE, STRIPE)])
    plsc.subcore_barrier()

    row0 = (ci * 16 + si) * (EP // 32 // 128)

    def block(b, _):
        rb = row0 + b * 8
        pltpu.sync_copy(dst2d.at[pl.ds(rb, 8)], dstb)
        pltpu.sync_copy(ew2d.at[pl.ds(rb, 8)], ewb)
        for j in range(8):
            pltpu.sync_copy(ewb.at[j], acc.at[dstb.at[j]], add=True)
        return 0

    lax.fori_loop(0, 25, block, 0)
    plsc.subcore_barrier()
    sl = pl.ds(si * STRIPE, STRIPE)

    @pl.when(ci == 0)
    def _():
        pltpu.sync_copy(acc.at[sl], dega.at[sl])

    @pl.when(ci == 1)
    def _():
        pltpu.sync_copy(acc.at[sl], degb.at[sl])


_deg_kernel = functools.partial(
    pl.kernel,
    out_type=(jax.ShapeDtypeStruct((NP,), jnp.float32),
              jax.ShapeDtypeStruct((NP,), jnp.float32)),
    mesh=_mesh,
    scratch_types=[
        pltpu.VMEM((8, 128), jnp.int32),
        pltpu.VMEM((8, 128), jnp.float32),
        pltpu.VMEM((STRIPE,), jnp.float32),
        pltpu.VMEM_SHARED((NP,), jnp.float32),
    ],
)(_deg_body)


def _make_agg(Ds, edge_split):
    KB = 8 if edge_split else 16
    NBLK = 25

    def body(htabA, htabB, initA, initB, src2d, dst2d, ew2d, outA, outB,
             srcb, dstb, ewb, rows, sem, acc):
        ci = lax.axis_index("c")
        si = lax.axis_index("s")
        sl = pl.ds(si * STRIPE, STRIPE)
        if edge_split:
            row0 = (ci * 16 + si) * (EP // 32 // 128)
        else:
            row0 = si * (EP // 16 // 128)

        def run(htab, init, out):
            pltpu.sync_copy(init.at[sl], acc.at[sl])
            plsc.subcore_barrier()

            def block(b, _):
                rb = row0 + b * KB
                pltpu.sync_copy(src2d.at[pl.ds(rb, KB)], srcb)
                pltpu.sync_copy(dst2d.at[pl.ds(rb, KB)], dstb)
                pltpu.sync_copy(ew2d.at[pl.ds(rb, KB)], ewb)
                for j in range(KB):
                    pltpu.async_copy(htab.at[srcb.at[j]], rows, sem).wait()

                    def grp(g, _):
                        wv = ewb[j, pl.ds(g * 16, 16)]
                        for l in range(16):
                            i = g * 16 + l
                            w = wv[l]
                            for f in range(Ds // 16):
                                rows[i, pl.ds(f * 16, 16)] = (
                                    rows[i, pl.ds(f * 16, 16)] * w)
                        return 0

                    lax.fori_loop(0, 8, grp, 0)
                    pltpu.sync_copy(rows, acc.at[dstb.at[j]], add=True)
                return 0

            lax.fori_loop(0, NBLK, block, 0)
            plsc.subcore_barrier()
            pltpu.sync_copy(acc.at[sl], out.at[sl])

        @pl.when(ci == 0)
        def _():
            run(htabA, initA, outA)

        @pl.when(ci == 1)
        def _():
            run(htabB, initB, outB)

    return functools.partial(
        pl.kernel,
        out_type=(jax.ShapeDtypeStruct((NP, Ds), jnp.float32),
                  jax.ShapeDtypeStruct((NP, Ds), jnp.float32)),
        mesh=_mesh,
        compiler_params=pltpu.CompilerParams(use_tc_tiling_on_sc=False),
        scratch_types=[
            pltpu.VMEM((KB, 128), jnp.int32),
            pltpu.VMEM((KB, 128), jnp.int32),
            pltpu.VMEM((KB, 128), jnp.float32),
            pltpu.VMEM((128, Ds), jnp.float32),
            pltpu.SemaphoreType.DMA,
            pltpu.VMEM_SHARED((NP, Ds), jnp.float32),
        ],
    )(body)


_agg32 = _make_agg(32, edge_split=False)
_agg16 = _make_agg(16, edge_split=False)
_agg16e = _make_agg(16, edge_split=True)


def _tc1_body(x_ref, w_ref, da_ref, db_ref, ha_ref, hb_ref, dinv_ref):
    deg = da_ref[0, 0, :] + db_ref[0, 0, :] + 1.0
    dinv = lax.rsqrt(deg)
    h = jnp.dot(x_ref[...], w_ref[...], preferred_element_type=jnp.float32)
    ht = h * dinv[:, None]
    ha_ref[...] = ht[:, :32]
    hb_ref[...] = ht[:, 32:]
    dinv_ref[0, 0, :] = dinv


def _tc1(xp, w1p, dega3, degb3):
    return pl.pallas_call(
        _tc1_body,
        grid=(GM,),
        in_specs=[
            pl.BlockSpec((BM, KP), lambda i: (i, 0)),
            pl.BlockSpec((KP, 64), lambda i: (0, 0)),
            pl.BlockSpec((1, 1, BM), lambda i: (i, 0, 0)),
            pl.BlockSpec((1, 1, BM), lambda i: (i, 0, 0)),
        ],
        out_specs=[
            pl.BlockSpec((BM, 32), lambda i: (i, 0)),
            pl.BlockSpec((BM, 32), lambda i: (i, 0)),
            pl.BlockSpec((1, 1, BM), lambda i: (i, 0, 0)),
        ],
        out_shape=[
            jax.ShapeDtypeStruct((NP, 32), jnp.float32),
            jax.ShapeDtypeStruct((NP, 32), jnp.float32),
            jax.ShapeDtypeStruct((GM, 1, BM), jnp.float32),
        ],
    )(xp, w1p, dega3, degb3)


def _tc2_body(aa_ref, ab_ref, dinv_ref, w_ref, b_ref, ha_ref, hb_ref):
    dinv = dinv_ref[0, 0, :]
    z = jnp.concatenate([aa_ref[...], ab_ref[...]], axis=1)
    z = jnp.maximum(z * dinv[:, None] + b_ref[...], 0.0)
    ht = jnp.dot(z, w_ref[...], preferred_element_type=jnp.float32)
    ht = ht * dinv[:, None]
    ha_ref[...] = ht[:, :16]
    hb_ref[...] = ht[:, 16:]


def _tc2(acc1a, acc1b, dinv3, w2, b1r):
    return pl.pallas_call(
        _tc2_body,
        grid=(GM,),
        in_specs=[
            pl.BlockSpec((BM, 32), lambda i: (i, 0)),
            pl.BlockSpec((BM, 32), lambda i: (i, 0)),
            pl.BlockSpec((1, 1, BM), lambda i: (i, 0, 0)),
            pl.BlockSpec((64, 32), lambda i: (0, 0)),
            pl.BlockSpec((1, 64), lambda i: (0, 0)),
        ],
        out_specs=[
            pl.BlockSpec((BM, 16), lambda i: (i, 0)),
            pl.BlockSpec((BM, 16), lambda i: (i, 0)),
        ],
        out_shape=[
            jax.ShapeDtypeStruct((NP, 16), jnp.float32),
            jax.ShapeDtypeStruct((NP, 16), jnp.float32),
        ],
    )(acc1a, acc1b, dinv3, w2, b1r)


def _tc3_body(aa_ref, ab_ref, dinv_ref, w_ref, b_ref, h_ref):
    dinv = dinv_ref[0, 0, :]
    z = jnp.concatenate([aa_ref[...], ab_ref[...]], axis=1)
    z = jnp.maximum(z * dinv[:, None] + b_ref[...], 0.0)
    ht = jnp.dot(z, w_ref[...], preferred_element_type=jnp.float32)
    h_ref[...] = ht * dinv[:, None]


def _tc3(acc2a, acc2b, dinv3, w3p, b2r):
    return pl.pallas_call(
        _tc3_body,
        grid=(GM,),
        in_specs=[
            pl.BlockSpec((BM, 16), lambda i: (i, 0)),
            pl.BlockSpec((BM, 16), lambda i: (i, 0)),
            pl.BlockSpec((1, 1, BM), lambda i: (i, 0, 0)),
            pl.BlockSpec((32, 16), lambda i: (0, 0)),
            pl.BlockSpec((1, 32), lambda i: (0, 0)),
        ],
        out_specs=pl.BlockSpec((BM, 16), lambda i: (i, 0)),
        out_shape=jax.ShapeDtypeStruct((NP, 16), jnp.float32),
    )(acc2a, acc2b, dinv3, w3p, b2r)


def _tc4_body(aa_ref, ab_ref, dinv_ref, b_ref, o_ref):
    dinv = dinv_ref[0, 0, :]
    o_ref[...] = (aa_ref[...] + ab_ref[...]) * dinv[:, None] + b_ref[...]


def _tc4(acc3a, acc3b, dinv3, b3r):
    return pl.pallas_call(
        _tc4_body,
        grid=(GM,),
        in_specs=[
            pl.BlockSpec((BM, 16), lambda i: (i, 0)),
            pl.BlockSpec((BM, 16), lambda i: (i, 0)),
            pl.BlockSpec((1, 1, BM), lambda i: (i, 0, 0)),
            pl.BlockSpec((1, 16), lambda i: (0, 0)),
        ],
        out_specs=pl.BlockSpec((BM, 16), lambda i: (i, 0)),
        out_shape=jax.ShapeDtypeStruct((NP, 16), jnp.float32),
    )(acc3a, acc3b, dinv3, b3r)


def kernel(x, edge_index, edge_weight, W1, b1, W2, b2, W3, b3):
    f32 = jnp.float32
    xp = jnp.pad(x, ((0, NP - N), (0, KP - x.shape[1])))
    w1p = jnp.pad(W1, ((0, KP - W1.shape[0]), (0, 0)))
    w3p = jnp.pad(W3, ((0, 0), (0, 16 - W3.shape[1])))
    b1r = b1.reshape(1, 64)
    b2r = b2.reshape(1, 32)
    b3r = jnp.broadcast_to(b3.reshape(1, 1), (1, 16))

    src2d = jnp.pad(edge_index[0], (0, EP - E)).reshape(EP // 128, 128)
    dst2d = jnp.pad(edge_index[1], (0, EP - E)).reshape(EP // 128, 128)
    ew2d = jnp.pad(edge_weight, (0, EP - E)).reshape(EP // 128, 128)

    dega, degb = _deg_kernel(dst2d, ew2d)
    dega3 = dega.reshape(GM, 1, BM)
    degb3 = degb.reshape(GM, 1, BM)

    h1a, h1b, dinv3 = _tc1(xp, w1p, dega3, degb3)
    acc1a, acc1b = _agg32(h1a, h1b, h1a, h1b, src2d, dst2d, ew2d)

    h2a, h2b = _tc2(acc1a, acc1b, dinv3, W2, b1r)
    acc2a, acc2b = _agg16(h2a, h2b, h2a, h2b, src2d, dst2d, ew2d)

    h3 = _tc3(acc2a, acc2b, dinv3, w3p, b2r)
    zeros16 = jnp.zeros((NP, 16), f32)
    acc3a, acc3b = _agg16e(h3, h3, h3, zeros16, src2d, dst2d, ew2d)

    outp = _tc4(acc3a, acc3b, dinv3, b3r)
    return outp[:N, 0]
